# baseline (speedup 1.0000x reference)
import functools

import jax
import jax.numpy as jnp
from jax import lax
from jax.experimental import pallas as pl
from jax.experimental.pallas import tpu as pltpu
from jax.experimental.pallas import tpu_sc as plsc

NUM_CODES = 8192
DIM = 256
TOKENS = 8192
COMMITMENT = 0.25

TT = 256
CT = 1024
NTILES = TOKENS // TT
NCHUNK = NUM_CODES // CT

SC_CORES = 2
SC_SUBCORES = 16
SC_WORKERS = SC_CORES * SC_SUBCORES
TOK_PER_W = TOKENS // SC_WORKERS
GCHUNK = 128
NGC = TOK_PER_W // GCHUNK


def _argmin_body(x_ref, a_ref, b_ref, emb_ref, idx_ref, dmin_ref):
    xb = x_ref[...].astype(jnp.bfloat16)
    a = a_ref[...]
    iota = lax.broadcasted_iota(jnp.int32, (TT, CT), 1)

    def chunk(k, carry):
        rv, ri = carry
        eb = emb_ref[pl.ds(k * CT, CT), :].astype(jnp.bfloat16)
        bb = b_ref[:, pl.ds(k * CT, CT)]
        c = lax.dot_general(xb, eb, (((1,), (1,)), ((), ())),
                            preferred_element_type=jnp.float32)
        score = bb - 2.0 * c
        upd = score < rv
        return jnp.where(upd, score, rv), jnp.where(upd, iota + k * CT, ri)

    rv0 = jnp.full((TT, CT), jnp.inf, jnp.float32)
    ri0 = jnp.zeros((TT, CT), jnp.int32)
    rv, ri = lax.fori_loop(0, NCHUNK, chunk, (rv0, ri0))
    bv = jnp.min(rv, axis=1)
    bi = jnp.min(jnp.where(rv == bv[:, None], ri, NUM_CODES), axis=1)
    idx_ref[0, 0, :] = bi
    dmin_ref[0, 0, :] = bv + a[:, 0]


def _argmin_call(x_flat, a, b, emb):
    return pl.pallas_call(
        _argmin_body,
        grid=(NTILES,),
        in_specs=[
            pl.BlockSpec((TT, DIM), lambda i: (i, 0)),
            pl.BlockSpec((TT, 1), lambda i: (i, 0)),
            pl.BlockSpec((1, NUM_CODES), lambda i: (0, 0)),
            pl.BlockSpec((NUM_CODES, DIM), lambda i: (0, 0)),
        ],
        out_specs=[
            pl.BlockSpec((1, 1, TT), lambda i: (i, 0, 0)),
            pl.BlockSpec((1, 1, TT), lambda i: (i, 0, 0)),
        ],
        out_shape=[
            jax.ShapeDtypeStruct((NTILES, 1, TT), jnp.int32),
            jax.ShapeDtypeStruct((NTILES, 1, TT), jnp.float32),
        ],
    )(x_flat, a, b, emb)


def _sc_body(idx_hbm, emb_hbm, out_hbm, counts_hbm,
             idx_v, rows_v, zeros_v, ones_v, counts_sh, sem):
    c = lax.axis_index("c")
    s = lax.axis_index("s")
    wid = c * SC_SUBCORES + s
    base = wid * TOK_PER_W

    for j in range(NGC):
        pltpu.sync_copy(idx_hbm.at[pl.ds(base + j * GCHUNK, GCHUNK)],
                        idx_v.at[j])
    copies = [pltpu.async_copy(emb_hbm.at[idx_v.at[j]], rows_v.at[j], sem)
              for j in range(NGC)]

    for t in range(512 // 16):
        zeros_v[pl.ds(t * 16, 16)] = jnp.zeros((16,), jnp.float32)
    for t in range(GCHUNK // 16):
        ones_v[pl.ds(t * 16, 16)] = jnp.ones((16,), jnp.float32)
    pltpu.sync_copy(zeros_v, counts_sh.at[pl.ds(s * 512, 512)])
    plsc.subcore_barrier()

    for j in range(NGC):
        pltpu.sync_copy(ones_v, counts_sh.at[idx_v.at[j]], add=True)

    for cp in copies:
        cp.wait()
    for j in range(NGC):
        pltpu.sync_copy(rows_v.at[j],
                        out_hbm.at[pl.ds(base + j * GCHUNK, GCHUNK)])

    plsc.subcore_barrier()

    @pl.when(s == 0)
    def _():
        pltpu.sync_copy(counts_sh, counts_hbm.at[c])


@functools.cache
def _sc_gather_hist():
    return functools.partial(
        pl.kernel,
        out_type=[
            jax.ShapeDtypeStruct((TOKENS, DIM), jnp.float32),
            jax.ShapeDtypeStruct((SC_CORES, NUM_CODES), jnp.float32),
        ],
        mesh=plsc.VectorSubcoreMesh(core_axis_name="c", subcore_axis_name="s"),
        scratch_types=[
            pltpu.VMEM((NGC, GCHUNK), jnp.int32),
            pltpu.VMEM((NGC, GCHUNK, DIM), jnp.float32),
            pltpu.VMEM((512,), jnp.float32),
            pltpu.VMEM((GCHUNK,), jnp.float32),
            pltpu.VMEM_SHARED((NUM_CODES,), jnp.float32),
            pltpu.SemaphoreType.DMA,
        ],
    )(_sc_body)


def _final_body(counts_ref, dmin_ref, loss_ref, perp_ref):
    counts = counts_ref[0, :] + counts_ref[1, :]
    avg = counts * (1.0 / TOKENS)
    ent = -jnp.sum(avg * jnp.log(avg + 1e-10))
    perp_ref[...] = jnp.exp(ent).reshape(1, 1)
    m = jnp.sum(dmin_ref[...]) / (TOKENS * DIM)
    loss_ref[...] = (m + COMMITMENT * m).reshape(1, 1)


def _final_call(counts, dmin):
    return pl.pallas_call(
        _final_body,
        out_shape=[
            jax.ShapeDtypeStruct((1, 1), jnp.float32),
            jax.ShapeDtypeStruct((1, 1), jnp.float32),
        ],
    )(counts, dmin)


def kernel(x, embedding):
    x_flat = x.reshape(-1, DIM)
    a = jnp.sum(x_flat ** 2, axis=1, keepdims=True)
    b = jnp.sum(embedding ** 2, axis=1).reshape(1, NUM_CODES)

    idx3, dmin3 = _argmin_call(x_flat, a, b, embedding)
    idx = idx3.reshape(TOKENS)

    quantized_flat, counts = _sc_gather_hist()(idx, embedding)
    loss2, perp2 = _final_call(counts, dmin3)

    quantized_st = quantized_flat.reshape(x.shape)
    return (quantized_st, loss2[0, 0], perp2[0, 0])


# speedup vs baseline: 1.0607x; 1.0607x over previous
import functools

import jax
import jax.numpy as jnp
from jax import lax
from jax.experimental import pallas as pl
from jax.experimental.pallas import tpu as pltpu
from jax.experimental.pallas import tpu_sc as plsc

NUM_CODES = 8192
DIM = 256
TOKENS = 8192
COMMITMENT = 0.25

TT = 256
CT = 1024
NTILES = TOKENS // TT
NCHUNK = NUM_CODES // CT

SC_CORES = 2
SC_SUBCORES = 16
SC_WORKERS = SC_CORES * SC_SUBCORES
TOK_PER_W = TOKENS // SC_WORKERS
GCHUNK = 128
NGC = TOK_PER_W // GCHUNK


def _argmin_body(x_ref, a_ref, b_ref, emb_ref, idx_ref, dmin_ref):
    xb = x_ref[...].astype(jnp.bfloat16)
    a = a_ref[...]
    iota = lax.broadcasted_iota(jnp.int32, (TT, CT), 1)

    def chunk(k, carry):
        bv, bi = carry
        eb = emb_ref[pl.ds(k * CT, CT), :].astype(jnp.bfloat16)
        bb = b_ref[:, pl.ds(k * CT, CT)]
        c = lax.dot_general(xb, eb, (((1,), (1,)), ((), ())),
                            preferred_element_type=jnp.float32)
        score = bb - 2.0 * c
        m = jnp.min(score, axis=1)
        lidx = jnp.min(jnp.where(score == m[:, None], iota, NUM_CODES),
                       axis=1) + k * CT
        upd = m < bv
        return jnp.where(upd, m, bv), jnp.where(upd, lidx, bi)

    bv0 = jnp.full((TT,), jnp.inf, jnp.float32)
    bi0 = jnp.zeros((TT,), jnp.int32)
    bv, bi = lax.fori_loop(0, NCHUNK, chunk, (bv0, bi0))
    idx_ref[0, 0, :] = bi
    dmin_ref[0, 0, :] = bv + a[:, 0]


def _argmin_call(x_flat, a, b, emb):
    return pl.pallas_call(
        _argmin_body,
        grid=(NTILES,),
        in_specs=[
            pl.BlockSpec((TT, DIM), lambda i: (i, 0)),
            pl.BlockSpec((TT, 1), lambda i: (i, 0)),
            pl.BlockSpec((1, NUM_CODES), lambda i: (0, 0)),
            pl.BlockSpec((NUM_CODES, DIM), lambda i: (0, 0)),
        ],
        out_specs=[
            pl.BlockSpec((1, 1, TT), lambda i: (i, 0, 0)),
            pl.BlockSpec((1, 1, TT), lambda i: (i, 0, 0)),
        ],
        out_shape=[
            jax.ShapeDtypeStruct((NTILES, 1, TT), jnp.int32),
            jax.ShapeDtypeStruct((NTILES, 1, TT), jnp.float32),
        ],
    )(x_flat, a, b, emb)


def _sc_body(idx_hbm, emb_hbm, out_hbm, counts_hbm,
             idx_v, rows_v, zeros_v, ones_v, counts_sh, sem):
    c = lax.axis_index("c")
    s = lax.axis_index("s")
    wid = c * SC_SUBCORES + s
    base = wid * TOK_PER_W

    for j in range(NGC):
        pltpu.sync_copy(idx_hbm.at[pl.ds(base + j * GCHUNK, GCHUNK)],
                        idx_v.at[j])
    copies = [pltpu.async_copy(emb_hbm.at[idx_v.at[j]], rows_v.at[j], sem)
              for j in range(NGC)]

    for t in range(512 // 16):
        zeros_v[pl.ds(t * 16, 16)] = jnp.zeros((16,), jnp.float32)
    for t in range(GCHUNK // 16):
        ones_v[pl.ds(t * 16, 16)] = jnp.ones((16,), jnp.float32)
    pltpu.sync_copy(zeros_v, counts_sh.at[pl.ds(s * 512, 512)])
    plsc.subcore_barrier()

    for j in range(NGC):
        pltpu.sync_copy(ones_v, counts_sh.at[idx_v.at[j]], add=True)

    for cp in copies:
        cp.wait()
    for j in range(NGC):
        pltpu.sync_copy(rows_v.at[j],
                        out_hbm.at[pl.ds(base + j * GCHUNK, GCHUNK)])

    plsc.subcore_barrier()

    @pl.when(s == 0)
    def _():
        pltpu.sync_copy(counts_sh, counts_hbm.at[c])


@functools.cache
def _sc_gather_hist():
    return functools.partial(
        pl.kernel,
        out_type=[
            jax.ShapeDtypeStruct((TOKENS, DIM), jnp.float32),
            jax.ShapeDtypeStruct((SC_CORES, NUM_CODES), jnp.float32),
        ],
        mesh=plsc.VectorSubcoreMesh(core_axis_name="c", subcore_axis_name="s"),
        scratch_types=[
            pltpu.VMEM((NGC, GCHUNK), jnp.int32),
            pltpu.VMEM((NGC, GCHUNK, DIM), jnp.float32),
            pltpu.VMEM((512,), jnp.float32),
            pltpu.VMEM((GCHUNK,), jnp.float32),
            pltpu.VMEM_SHARED((NUM_CODES,), jnp.float32),
            pltpu.SemaphoreType.DMA,
        ],
    )(_sc_body)


def _final_body(counts_ref, dmin_ref, loss_ref, perp_ref):
    counts = counts_ref[0, :] + counts_ref[1, :]
    avg = counts * (1.0 / TOKENS)
    ent = -jnp.sum(avg * jnp.log(avg + 1e-10))
    perp_ref[...] = jnp.exp(ent).reshape(1, 1)
    m = jnp.sum(dmin_ref[...]) / (TOKENS * DIM)
    loss_ref[...] = (m + COMMITMENT * m).reshape(1, 1)


def _final_call(counts, dmin):
    return pl.pallas_call(
        _final_body,
        out_shape=[
            jax.ShapeDtypeStruct((1, 1), jnp.float32),
            jax.ShapeDtypeStruct((1, 1), jnp.float32),
        ],
    )(counts, dmin)


def kernel(x, embedding):
    x_flat = x.reshape(-1, DIM)
    a = jnp.sum(x_flat ** 2, axis=1, keepdims=True)
    b = jnp.sum(embedding ** 2, axis=1).reshape(1, NUM_CODES)

    idx3, dmin3 = _argmin_call(x_flat, a, b, embedding)
    idx = idx3.reshape(TOKENS)

    quantized_flat, counts = _sc_gather_hist()(idx, embedding)
    loss2, perp2 = _final_call(counts, dmin3)

    quantized_st = quantized_flat.reshape(x.shape)
    return (quantized_st, loss2[0, 0], perp2[0, 0])
